# initial kernel scaffold (unmeasured)
import functools

import jax
import jax.numpy as jnp
from jax import lax
from jax.experimental import pallas as pl
from jax.experimental.pallas import tpu as pltpu

N_DEV = 4
SQ = 2048
SKV_SH = 2048
H = 32
H_LOC = 8
DH = 128
DM = 1024
SCALE = 0.08838834764831843
Q_TILE = 256

_MESH = pl.DeviceIdType.MESH


def _qproj(x, Wq):

    def body(x_ref, wq_ref, q_ref):
        xb = x_ref[0].astype(jnp.bfloat16)
        wb = wq_ref[...].astype(jnp.bfloat16)
        q_ref[0] = jnp.dot(xb, wb, preferred_element_type=jnp.float32).astype(
            jnp.bfloat16
        )

    return pl.pallas_call(
        body,
        grid=(H_LOC,),
        in_specs=[
            pl.BlockSpec((1, SQ, DM), lambda h: (0, 0, 0)),
            pl.BlockSpec((DM, DH), lambda h: (0, h)),
        ],
        out_specs=pl.BlockSpec((1, SQ, DH), lambda h: (h, 0, 0)),
        out_shape=jax.ShapeDtypeStruct((H_LOC, SQ, DH), jnp.bfloat16),
    )(x, Wq)


def _kv_headmajor(K_ext, V_ext):

    def body(k_ref, v_ref, kt_ref, vt_ref):
        kt_ref[0] = k_ref[0, :, 0, :].astype(jnp.bfloat16)
        vt_ref[0] = v_ref[0, :, 0, :].astype(jnp.bfloat16)

    return pl.pallas_call(
        body,
        grid=(H,),
        in_specs=[
            pl.BlockSpec((1, SKV_SH, 1, DH), lambda h: (0, 0, h, 0)),
            pl.BlockSpec((1, SKV_SH, 1, DH), lambda h: (0, 0, h, 0)),
        ],
        out_specs=[
            pl.BlockSpec((1, SKV_SH, DH), lambda h: (h, 0, 0)),
            pl.BlockSpec((1, SKV_SH, DH), lambda h: (h, 0, 0)),
        ],
        out_shape=[
            jax.ShapeDtypeStruct((H, SKV_SH, DH), jnp.bfloat16),
            jax.ShapeDtypeStruct((H, SKV_SH, DH), jnp.bfloat16),
        ],
    )(K_ext, V_ext)


def _exchange(K_t, V_t):

    def body(kt_ref, vt_ref, kg_ref, vg_ref, send_sems, recv_sems, copy_sems):
        my = lax.axis_index("i")

        bsem = pltpu.get_barrier_semaphore()
        for d in range(1, N_DEV):
            peer = lax.rem(my + d, N_DEV)
            pl.semaphore_signal(bsem, inc=1, device_id=(peer,), device_id_type=_MESH)
        pl.semaphore_wait(bsem, N_DEV - 1)

        loc_k = pltpu.make_async_copy(
            kt_ref.at[pl.ds(my * H_LOC, H_LOC)],
            kg_ref.at[:, pl.ds(my * SKV_SH, SKV_SH), :],
            copy_sems.at[0],
        )
        loc_v = pltpu.make_async_copy(
            vt_ref.at[pl.ds(my * H_LOC, H_LOC)],
            vg_ref.at[:, pl.ds(my * SKV_SH, SKV_SH), :],
            copy_sems.at[1],
        )
        loc_k.start()
        loc_v.start()

        sends = []
        for d in range(1, N_DEV):
            peer = lax.rem(my + d, N_DEV)
            k_rdma = pltpu.make_async_remote_copy(
                src_ref=kt_ref.at[pl.ds(peer * H_LOC, H_LOC)],
                dst_ref=kg_ref.at[:, pl.ds(my * SKV_SH, SKV_SH), :],
                send_sem=send_sems.at[d - 1],
                recv_sem=recv_sems.at[d - 1],
                device_id=(peer,),
                device_id_type=_MESH,
            )
            v_rdma = pltpu.make_async_remote_copy(
                src_ref=vt_ref.at[pl.ds(peer * H_LOC, H_LOC)],
                dst_ref=vg_ref.at[:, pl.ds(my * SKV_SH, SKV_SH), :],
                send_sem=send_sems.at[3 + d - 1],
                recv_sem=recv_sems.at[3 + d - 1],
                device_id=(peer,),
                device_id_type=_MESH,
            )
            k_rdma.start()
            v_rdma.start()
            sends.append((k_rdma, v_rdma))

        for d in range(1, N_DEV):
            src = lax.rem(my - d + N_DEV, N_DEV)
            k_recv = pltpu.make_async_remote_copy(
                src_ref=kt_ref.at[pl.ds(0, H_LOC)],
                dst_ref=kg_ref.at[:, pl.ds(src * SKV_SH, SKV_SH), :],
                send_sem=send_sems.at[d - 1],
                recv_sem=recv_sems.at[d - 1],
                device_id=(src,),
                device_id_type=_MESH,
            )
            v_recv = pltpu.make_async_remote_copy(
                src_ref=vt_ref.at[pl.ds(0, H_LOC)],
                dst_ref=vg_ref.at[:, pl.ds(src * SKV_SH, SKV_SH), :],
                send_sem=send_sems.at[3 + d - 1],
                recv_sem=recv_sems.at[3 + d - 1],
                device_id=(src,),
                device_id_type=_MESH,
            )
            k_recv.wait_recv()
            v_recv.wait_recv()

        for k_rdma, v_rdma in sends:
            k_rdma.wait_send()
            v_rdma.wait_send()
        loc_k.wait()
        loc_v.wait()

    return pl.pallas_call(
        body,
        in_specs=[
            pl.BlockSpec(memory_space=pltpu.MemorySpace.ANY),
            pl.BlockSpec(memory_space=pltpu.MemorySpace.ANY),
        ],
        out_specs=[
            pl.BlockSpec(memory_space=pltpu.MemorySpace.ANY),
            pl.BlockSpec(memory_space=pltpu.MemorySpace.ANY),
        ],
        out_shape=[
            jax.ShapeDtypeStruct((H_LOC, N_DEV * SKV_SH, DH), jnp.bfloat16),
            jax.ShapeDtypeStruct((H_LOC, N_DEV * SKV_SH, DH), jnp.bfloat16),
        ],
        scratch_shapes=[
            pltpu.SemaphoreType.DMA((6,)),
            pltpu.SemaphoreType.DMA((6,)),
            pltpu.SemaphoreType.DMA((2,)),
        ],
        compiler_params=pltpu.CompilerParams(collective_id=0),
    )(K_t, V_t)


def _attention(Q_t, K_g, V_g):
    SKV = N_DEV * SKV_SH

    def body(q_ref, k_ref, v_ref, o_ref):
        qi = pl.program_id(1)
        q = q_ref[0]
        k = k_ref[0]
        s = lax.dot_general(
            q, k, (((1,), (1,)), ((), ())), preferred_element_type=jnp.float32
        )
        s = s * SCALE
        row = lax.broadcasted_iota(jnp.int32, (Q_TILE, SKV), 0)
        col = lax.broadcasted_iota(jnp.int32, (Q_TILE, SKV), 1)
        qb = (qi * Q_TILE + row) // 64
        kb = col // 64
        keep = (qb == kb) | (kb == 0) | (((qb + kb) % 3) == 0)
        s = jnp.where(keep, s, -1e9)
        m = jnp.max(s, axis=1, keepdims=True)
        p = jnp.exp(s - m)
        l = jnp.sum(p, axis=1, keepdims=True)
        ctx = jnp.dot(
            p.astype(jnp.bfloat16), v_ref[0], preferred_element_type=jnp.float32
        )
        o_ref[:, 0, :] = (ctx / l).astype(jnp.bfloat16)

    return pl.pallas_call(
        body,
        grid=(H_LOC, SQ // Q_TILE),
        in_specs=[
            pl.BlockSpec((1, Q_TILE, DH), lambda h, qi: (h, qi, 0)),
            pl.BlockSpec((1, SKV, DH), lambda h, qi: (h, 0, 0)),
            pl.BlockSpec((1, SKV, DH), lambda h, qi: (h, 0, 0)),
        ],
        out_specs=pl.BlockSpec((Q_TILE, 1, DH), lambda h, qi: (qi, h, 0)),
        out_shape=jax.ShapeDtypeStruct((SQ, H_LOC, DH), jnp.bfloat16),
    )(Q_t, K_g, V_g)


def _out_allreduce(ctx, Wo):

    def body(ctx_ref, wo_ref, o_ref, send_buf, recv_buf, send_sems, recv_sems):
        my = lax.axis_index("i")
        part = jnp.dot(
            ctx_ref[...],
            wo_ref[...].astype(jnp.bfloat16),
            preferred_element_type=jnp.float32,
        )
        send_buf[...] = part.astype(jnp.bfloat16)

        bsem = pltpu.get_barrier_semaphore()
        for d in range(1, N_DEV):
            peer = lax.rem(my + d, N_DEV)
            pl.semaphore_signal(bsem, inc=1, device_id=(peer,), device_id_type=_MESH)
        pl.semaphore_wait(bsem, N_DEV - 1)

        sends = []
        for d in range(1, N_DEV):
            peer = lax.rem(my + d, N_DEV)
            rdma = pltpu.make_async_remote_copy(
                src_ref=send_buf,
                dst_ref=recv_buf.at[d - 1],
                send_sem=send_sems.at[d - 1],
                recv_sem=recv_sems.at[d - 1],
                device_id=(peer,),
                device_id_type=_MESH,
            )
            rdma.start()
            sends.append(rdma)

        acc = part
        for d in range(1, N_DEV):
            src = lax.rem(my - d + N_DEV, N_DEV)
            recv = pltpu.make_async_remote_copy(
                src_ref=send_buf,
                dst_ref=recv_buf.at[d - 1],
                send_sem=send_sems.at[d - 1],
                recv_sem=recv_sems.at[d - 1],
                device_id=(src,),
                device_id_type=_MESH,
            )
            recv.wait_recv()
            acc = acc + recv_buf[d - 1].astype(jnp.float32)
        o_ref[0] = acc

        for rdma in sends:
            rdma.wait_send()

    return pl.pallas_call(
        body,
        in_specs=[
            pl.BlockSpec((SQ, DM), lambda: (0, 0)),
            pl.BlockSpec((DM, DM), lambda: (0, 0)),
        ],
        out_specs=pl.BlockSpec((1, SQ, DM), lambda: (0, 0, 0)),
        out_shape=jax.ShapeDtypeStruct((1, SQ, DM), jnp.float32),
        scratch_shapes=[
            pltpu.VMEM((SQ, DM), jnp.bfloat16),
            pltpu.VMEM((N_DEV - 1, SQ, DM), jnp.bfloat16),
            pltpu.SemaphoreType.DMA((3,)),
            pltpu.SemaphoreType.DMA((3,)),
        ],
        compiler_params=pltpu.CompilerParams(collective_id=1),
    )(ctx, Wo)


def kernel(x, Wq, K_ext, V_ext, Wo):
    Q_t = _qproj(x, Wq)
    K_t, V_t = _kv_headmajor(K_ext, V_ext)
    K_g, V_g = _exchange(K_t, V_t)
    ctx = _attention(Q_t, K_g, V_g)
    out = _out_allreduce(ctx.reshape(SQ, H_LOC * DH), Wo)
    return out


# baseline (device time: 1047081 ns/iter reference)
import functools

import jax
import jax.numpy as jnp
from jax import lax
from jax.experimental import pallas as pl
from jax.experimental.pallas import tpu as pltpu

N_DEV = 4
SQ = 2048
SKV_SH = 2048
H = 32
H_LOC = 8
DH = 128
DM = 1024
SCALE = 0.08838834764831843
Q_TILE = 256

_MESH = pl.DeviceIdType.MESH


def _qproj(x, Wq):

    def body(x_ref, wq_ref, q_ref):
        xb = x_ref[0].astype(jnp.bfloat16)
        wb = wq_ref[...].astype(jnp.bfloat16)
        q_ref[0] = jnp.dot(xb, wb, preferred_element_type=jnp.float32).astype(
            jnp.bfloat16
        )

    return pl.pallas_call(
        body,
        grid=(H_LOC,),
        in_specs=[
            pl.BlockSpec((1, SQ, DM), lambda h: (0, 0, 0)),
            pl.BlockSpec((DM, DH), lambda h: (0, h)),
        ],
        out_specs=pl.BlockSpec((1, SQ, DH), lambda h: (h, 0, 0)),
        out_shape=jax.ShapeDtypeStruct((H_LOC, SQ, DH), jnp.bfloat16),
    )(x, Wq)


def _kv_headmajor(K_ext, V_ext):

    def body(k_ref, v_ref, kt_ref, vt_ref):
        kt_ref[0] = k_ref[...].astype(jnp.bfloat16)
        vt_ref[0] = v_ref[...].astype(jnp.bfloat16)

    return pl.pallas_call(
        body,
        grid=(H,),
        in_specs=[
            pl.BlockSpec((SKV_SH, DH), lambda h: (0, h)),
            pl.BlockSpec((SKV_SH, DH), lambda h: (0, h)),
        ],
        out_specs=[
            pl.BlockSpec((1, SKV_SH, DH), lambda h: (h, 0, 0)),
            pl.BlockSpec((1, SKV_SH, DH), lambda h: (h, 0, 0)),
        ],
        out_shape=[
            jax.ShapeDtypeStruct((H, SKV_SH, DH), jnp.bfloat16),
            jax.ShapeDtypeStruct((H, SKV_SH, DH), jnp.bfloat16),
        ],
    )(K_ext, V_ext)


def _exchange(K_t, V_t):

    def body(kt_ref, vt_ref, kg_ref, vg_ref, send_sems, recv_sems, copy_sems):
        my = lax.axis_index("i")

        bsem = pltpu.get_barrier_semaphore()
        for d in range(1, N_DEV):
            peer = lax.rem(my + d, N_DEV)
            pl.semaphore_signal(bsem, inc=1, device_id=(peer,), device_id_type=_MESH)
        pl.semaphore_wait(bsem, N_DEV - 1)

        loc_k = pltpu.make_async_copy(
            kt_ref.at[pl.ds(my * H_LOC, H_LOC)],
            kg_ref.at[:, pl.ds(my * SKV_SH, SKV_SH), :],
            copy_sems.at[0],
        )
        loc_v = pltpu.make_async_copy(
            vt_ref.at[pl.ds(my * H_LOC, H_LOC)],
            vg_ref.at[:, pl.ds(my * SKV_SH, SKV_SH), :],
            copy_sems.at[1],
        )
        loc_k.start()
        loc_v.start()

        sends = []
        for d in range(1, N_DEV):
            peer = lax.rem(my + d, N_DEV)
            k_rdma = pltpu.make_async_remote_copy(
                src_ref=kt_ref.at[pl.ds(peer * H_LOC, H_LOC)],
                dst_ref=kg_ref.at[:, pl.ds(my * SKV_SH, SKV_SH), :],
                send_sem=send_sems.at[d - 1],
                recv_sem=recv_sems.at[d - 1],
                device_id=(peer,),
                device_id_type=_MESH,
            )
            v_rdma = pltpu.make_async_remote_copy(
                src_ref=vt_ref.at[pl.ds(peer * H_LOC, H_LOC)],
                dst_ref=vg_ref.at[:, pl.ds(my * SKV_SH, SKV_SH), :],
                send_sem=send_sems.at[3 + d - 1],
                recv_sem=recv_sems.at[3 + d - 1],
                device_id=(peer,),
                device_id_type=_MESH,
            )
            k_rdma.start()
            v_rdma.start()
            sends.append((k_rdma, v_rdma))

        for d in range(1, N_DEV):
            src = lax.rem(my - d + N_DEV, N_DEV)
            k_recv = pltpu.make_async_remote_copy(
                src_ref=kt_ref.at[pl.ds(0, H_LOC)],
                dst_ref=kg_ref.at[:, pl.ds(src * SKV_SH, SKV_SH), :],
                send_sem=send_sems.at[d - 1],
                recv_sem=recv_sems.at[d - 1],
                device_id=(src,),
                device_id_type=_MESH,
            )
            v_recv = pltpu.make_async_remote_copy(
                src_ref=vt_ref.at[pl.ds(0, H_LOC)],
                dst_ref=vg_ref.at[:, pl.ds(src * SKV_SH, SKV_SH), :],
                send_sem=send_sems.at[3 + d - 1],
                recv_sem=recv_sems.at[3 + d - 1],
                device_id=(src,),
                device_id_type=_MESH,
            )
            k_recv.wait_recv()
            v_recv.wait_recv()

        for k_rdma, v_rdma in sends:
            k_rdma.wait_send()
            v_rdma.wait_send()
        loc_k.wait()
        loc_v.wait()

    return pl.pallas_call(
        body,
        in_specs=[
            pl.BlockSpec(memory_space=pltpu.MemorySpace.HBM),
            pl.BlockSpec(memory_space=pltpu.MemorySpace.HBM),
        ],
        out_specs=[
            pl.BlockSpec(memory_space=pltpu.MemorySpace.HBM),
            pl.BlockSpec(memory_space=pltpu.MemorySpace.HBM),
        ],
        out_shape=[
            jax.ShapeDtypeStruct((H_LOC, N_DEV * SKV_SH, DH), jnp.bfloat16),
            jax.ShapeDtypeStruct((H_LOC, N_DEV * SKV_SH, DH), jnp.bfloat16),
        ],
        scratch_shapes=[
            pltpu.SemaphoreType.DMA((6,)),
            pltpu.SemaphoreType.DMA((6,)),
            pltpu.SemaphoreType.DMA((2,)),
        ],
        compiler_params=pltpu.CompilerParams(collective_id=0),
    )(K_t, V_t)


def _attention(Q_t, K_g, V_g):
    SKV = N_DEV * SKV_SH

    def body(q_ref, k_ref, v_ref, o_ref):
        qi = pl.program_id(1)
        q = q_ref[0]
        k = k_ref[0]
        s = lax.dot_general(
            q, k, (((1,), (1,)), ((), ())), preferred_element_type=jnp.float32
        )
        s = s * SCALE
        row = lax.broadcasted_iota(jnp.int32, (Q_TILE, SKV), 0)
        col = lax.broadcasted_iota(jnp.int32, (Q_TILE, SKV), 1)
        qb = (qi * Q_TILE + row) // 64
        kb = col // 64
        keep = (qb == kb) | (kb == 0) | (((qb + kb) % 3) == 0)
        s = jnp.where(keep, s, -1e9)
        m = jnp.max(s, axis=1, keepdims=True)
        p = jnp.exp(s - m)
        l = jnp.sum(p, axis=1, keepdims=True)
        ctx = jnp.dot(
            p.astype(jnp.bfloat16), v_ref[0], preferred_element_type=jnp.float32
        )
        o_ref[0] = (ctx / l).astype(jnp.bfloat16)

    return pl.pallas_call(
        body,
        grid=(H_LOC, SQ // Q_TILE),
        in_specs=[
            pl.BlockSpec((1, Q_TILE, DH), lambda h, qi: (h, qi, 0)),
            pl.BlockSpec((1, SKV, DH), lambda h, qi: (h, 0, 0)),
            pl.BlockSpec((1, SKV, DH), lambda h, qi: (h, 0, 0)),
        ],
        out_specs=pl.BlockSpec((1, Q_TILE, DH), lambda h, qi: (h, qi, 0)),
        out_shape=jax.ShapeDtypeStruct((H_LOC, SQ, DH), jnp.bfloat16),
    )(Q_t, K_g, V_g)


def _out_allreduce(ctx, Wo):

    def body(ctx_ref, wo_ref, o_ref, send_buf, recv_buf, send_sems, recv_sems):
        my = lax.axis_index("i")
        part = jnp.zeros((SQ, DM), jnp.float32)
        for h in range(H_LOC):
            part = part + jnp.dot(
                ctx_ref[h],
                wo_ref[h * DH : (h + 1) * DH, :].astype(jnp.bfloat16),
                preferred_element_type=jnp.float32,
            )
        send_buf[...] = part.astype(jnp.bfloat16)

        bsem = pltpu.get_barrier_semaphore()
        for d in range(1, N_DEV):
            peer = lax.rem(my + d, N_DEV)
            pl.semaphore_signal(bsem, inc=1, device_id=(peer,), device_id_type=_MESH)
        pl.semaphore_wait(bsem, N_DEV - 1)

        sends = []
        for d in range(1, N_DEV):
            peer = lax.rem(my + d, N_DEV)
            rdma = pltpu.make_async_remote_copy(
                src_ref=send_buf,
                dst_ref=recv_buf.at[d - 1],
                send_sem=send_sems.at[d - 1],
                recv_sem=recv_sems.at[d - 1],
                device_id=(peer,),
                device_id_type=_MESH,
            )
            rdma.start()
            sends.append(rdma)

        acc = part
        for d in range(1, N_DEV):
            src = lax.rem(my - d + N_DEV, N_DEV)
            recv = pltpu.make_async_remote_copy(
                src_ref=send_buf,
                dst_ref=recv_buf.at[d - 1],
                send_sem=send_sems.at[d - 1],
                recv_sem=recv_sems.at[d - 1],
                device_id=(src,),
                device_id_type=_MESH,
            )
            recv.wait_recv()
            acc = acc + recv_buf[d - 1].astype(jnp.float32)
        o_ref[0] = acc

        for rdma in sends:
            rdma.wait_send()

    return pl.pallas_call(
        body,
        in_specs=[
            pl.BlockSpec((H_LOC, SQ, DH), lambda: (0, 0, 0)),
            pl.BlockSpec((DM, DM), lambda: (0, 0)),
        ],
        out_specs=pl.BlockSpec((1, SQ, DM), lambda: (0, 0, 0)),
        out_shape=jax.ShapeDtypeStruct((1, SQ, DM), jnp.float32),
        scratch_shapes=[
            pltpu.VMEM((SQ, DM), jnp.bfloat16),
            pltpu.VMEM((N_DEV - 1, SQ, DM), jnp.bfloat16),
            pltpu.SemaphoreType.DMA((3,)),
            pltpu.SemaphoreType.DMA((3,)),
        ],
        compiler_params=pltpu.CompilerParams(collective_id=1),
    )(ctx, Wo)


def kernel(x, Wq, K_ext, V_ext, Wo):
    Q_t = _qproj(x, Wq)
    K_t, V_t = _kv_headmajor(
        K_ext.reshape(SKV_SH, H * DH), V_ext.reshape(SKV_SH, H * DH)
    )
    K_g, V_g = _exchange(K_t, V_t)
    ctx = _attention(Q_t, K_g, V_g)
    out = _out_allreduce(ctx, Wo)
    return out
